# baseline (speedup 1.0000x reference)
import functools

import jax
import jax.numpy as jnp
from jax import lax
from jax.experimental import pallas as pl
from jax.experimental.pallas import tpu as pltpu
from jax.experimental.pallas import tpu_sc as plsc

SEQ_LEN = 2048
BATCH = 16
HIDDEN = 1024
HALF = HIDDEN // 2


def _body(table_hbm, lengths_hbm, out_hbm, len_s, sem):
    pltpu.sync_copy(lengths_hbm, len_s)
    cid = lax.axis_index("c")

    @pl.when(cid == 0)
    def _fw():
        def issue(b, carry):
            ln = len_s[b]
            pltpu.async_copy(
                table_hbm.at[(ln - 1) * BATCH + b, pl.ds(0, HALF)],
                out_hbm.at[b, pl.ds(0, HALF)], sem)
            return carry

        lax.fori_loop(0, BATCH, issue, 0)

    @pl.when(cid == 1)
    def _bw():
        def issue(b, carry):
            ln = len_s[b]
            pltpu.async_copy(
                table_hbm.at[(SEQ_LEN - ln) * BATCH + b, pl.ds(HALF, HALF)],
                out_hbm.at[b, pl.ds(HALF, HALF)], sem)
            return carry

        lax.fori_loop(0, BATCH, issue, 0)

    pltpu.make_async_copy(
        table_hbm.at[pl.ds(0, BATCH), pl.ds(0, HALF)],
        out_hbm.at[:, pl.ds(0, HALF)], sem).wait()


_gather = functools.partial(
    pl.kernel,
    out_type=jax.ShapeDtypeStruct((BATCH, HIDDEN), jnp.float32),
    mesh=plsc.ScalarSubcoreMesh(axis_name="c", num_cores=2),
    scratch_types=[
        pltpu.SMEM((BATCH,), jnp.int32),
        pltpu.SemaphoreType.DMA,
    ],
    compiler_params=pltpu.CompilerParams(needs_layout_passes=False),
)(_body)


@jax.jit
def kernel(lstm_out, lengths):
    table = lstm_out.reshape(SEQ_LEN * BATCH, HIDDEN)
    return _gather(table, lengths.astype(jnp.int32))
